# Initial kernel scaffold; baseline (speedup 1.0000x reference)
import jax
import jax.numpy as jnp
from jax.experimental import pallas as pl


def kernel(probs_batch, span_indices_batch):
    raise NotImplementedError("write your pallas kernel here")


# speedup vs baseline: 345.1897x; 345.1897x over previous
import jax
import jax.numpy as jnp
from jax import lax
from jax.experimental import pallas as pl
from jax.experimental.pallas import tpu as pltpu
from jax.experimental.pallas import tpu_sc as plsc

B = 4
C = 8000
NV = C // 16
NSPAN = 1000
THR = 0.5
KEY_BIAS = 0x7FFFFFFF


def _nms_body(sc_hbm, st_hbm, en_hbm,
              ks_hbm, kp_hbm, ss_hbm, es_hbm, lb_hbm,
              sc_v, scs_v, key_a, idx_a, key_b, idx_b,
              st_in, en_in, ss_v, es_v, lb_v, ks_v, kp_v,
              hist, pos, cov, pref):
  wid = lax.axis_index("s") * 2 + lax.axis_index("c")
  lane = lax.iota(jnp.int32, 16)

  @pl.when(wid < B)
  def _():
    b = wid
    pltpu.sync_copy(sc_hbm.at[b], sc_v)
    pltpu.sync_copy(st_hbm.at[b], st_in)
    pltpu.sync_copy(en_hbm.at[b], en_in)

    def init_i(i, _):
      s = sc_v[pl.ds(i * 16, 16)]
      key_a[pl.ds(i * 16, 16)] = KEY_BIAS - lax.bitcast_convert_type(s, jnp.int32)
      idx_a[pl.ds(i * 16, 16)] = lane + i * 16
      return 0
    lax.fori_loop(0, NV, init_i, 0)

    bufs = [(key_a, idx_a, key_b, idx_b), (key_b, idx_b, key_a, idx_a),
            (key_a, idx_a, key_b, idx_b), (key_b, idx_b, key_a, idx_a)]
    for p, (sk, si, dk, di) in enumerate(bufs):
      shift = 8 * p

      def zero_h(j, _):
        hist[pl.ds(j * 16, 16)] = jnp.zeros((16,), jnp.int32)
        return 0
      lax.fori_loop(0, 16, zero_h, 0)

      def hist_i(i, _, sk=sk, shift=shift):
        k = sk[pl.ds(i * 16, 16)]
        d = lax.shift_right_logical(k, shift) & 255
        cnt, last = plsc.scan_count(d)
        plsc.addupdate_scatter(hist, [d], cnt, mask=last)
        return 0
      lax.fori_loop(0, NV, hist_i, 0)

      def scan_h(j, carry):
        h = hist[pl.ds(j * 16, 16)]
        c = plsc.cumsum(h)
        pos[pl.ds(j * 16, 16)] = c - h + carry
        return carry + jnp.max(c)
      lax.fori_loop(0, 16, scan_h, jnp.int32(0))

      def perm_i(i, _, sk=sk, si=si, dk=dk, di=di, shift=shift):
        k = sk[pl.ds(i * 16, 16)]
        ii = si[pl.ds(i * 16, 16)]
        d = lax.shift_right_logical(k, shift) & 255
        cnt, last = plsc.scan_count(d)
        base = plsc.load_gather(pos, [d])
        tgt = base + cnt - 1
        plsc.store_scatter(dk, [tgt], k)
        plsc.store_scatter(di, [tgt], ii)
        plsc.addupdate_scatter(pos, [d], cnt, mask=last)
        return 0
      lax.fori_loop(0, NV, perm_i, 0)

    def post_i(i, _):
      ii = idx_a[pl.ds(i * 16, 16)]
      k = key_a[pl.ds(i * 16, 16)]
      sidx = lax.shift_right_logical(ii, 3)
      ss_v[pl.ds(i * 16, 16)] = plsc.load_gather(st_in, [sidx])
      es_v[pl.ds(i * 16, 16)] = plsc.load_gather(en_in, [sidx])
      lb_v[pl.ds(i * 16, 16)] = ii & 7
      scs_v[pl.ds(i * 16, 16)] = lax.bitcast_convert_type(KEY_BIAS - k, jnp.float32)
      ks_v[pl.ds(i * 16, 16)] = jnp.zeros((16,), jnp.float32)
      kp_v[pl.ds(i * 16, 16)] = jnp.zeros((16,), jnp.int32)
      return 0
    lax.fori_loop(0, NV, post_i, 0)

    def zero_cp(j, _):
      cov[pl.ds(j * 16, 16)] = jnp.zeros((16,), jnp.int32)
      pref[pl.ds(j * 16, 16)] = jnp.zeros((16,), jnp.int32)
      return 0
    lax.fori_loop(0, 32, zero_cp, 0)

    def g_cond(cr):
      g, cont = cr
      return jnp.logical_and(g < NV, cont)

    def g_body(cr):
      g, _ = cr
      off = g * 16
      scv = scs_v[pl.ds(off, 16)]
      anyv = jnp.max(scv) > THR

      @pl.when(anyv)
      def _():
        valid = scv > THR
        st = ss_v[pl.ds(off, 16)]
        en = es_v[pl.ds(off, 16)]

        def i_cond(ic):
          return ic[1]

        def i_body(ic):
          prev_k, _, kvec = ic
          pfs = plsc.load_gather(pref, [st])
          pfe = plsc.load_gather(pref, [en])
          cvs = plsc.load_gather(cov, [st])
          conf = (pfe - pfs + cvs) > 0
          cand = jnp.logical_and(
              jnp.logical_and(valid, jnp.logical_not(conf)), lane > prev_k)
          npop = jnp.max(plsc.all_reduce_population_count(cand))
          has = npop > 0
          kidx = jnp.max(plsc.all_reduce_ffs(cand))

          @pl.when(has)
          def _():
            onehot = lane == kidx
            s_k = jnp.max(jnp.where(onehot, st, jnp.int32(-1)))
            e_k = jnp.max(jnp.where(onehot, en, jnp.int32(-1)))

            def upd(j, carry):
              gpos = lane + j * 16
              cvj = cov[pl.ds(j * 16, 16)]
              m = jnp.logical_and(gpos >= s_k, gpos <= e_k)
              cvj = jnp.where(m, jnp.int32(1), cvj)
              cov[pl.ds(j * 16, 16)] = cvj
              cs = plsc.cumsum(cvj)
              pref[pl.ds(j * 16, 16)] = cs + carry
              return carry + jnp.max(cs)
            lax.fori_loop(0, 32, upd, jnp.int32(0))

          kvec2 = jnp.where(jnp.logical_and(has, lane == kidx),
                            jnp.int32(1), kvec)
          prev2 = jnp.where(has, kidx, prev_k)
          return (prev2, has, kvec2)

        _, _, kfin = lax.while_loop(
            i_cond, i_body,
            (jnp.int32(-1), True, jnp.zeros((16,), jnp.int32)))
        kp_v[pl.ds(off, 16)] = kfin
        ks_v[pl.ds(off, 16)] = scv * kfin.astype(jnp.float32)

      return (g + 1, anyv)

    lax.while_loop(g_cond, g_body, (jnp.int32(0), True))

    pltpu.sync_copy(ks_v, ks_hbm.at[b])
    pltpu.sync_copy(kp_v, kp_hbm.at[b])
    pltpu.sync_copy(ss_v, ss_hbm.at[b])
    pltpu.sync_copy(es_v, es_hbm.at[b])
    pltpu.sync_copy(lb_v, lb_hbm.at[b])


def kernel(probs_batch, span_indices_batch):
  sc = probs_batch.reshape(B, C)
  st = span_indices_batch[..., 0]
  en = span_indices_batch[..., 1]

  mesh = plsc.VectorSubcoreMesh(core_axis_name="c", subcore_axis_name="s")
  out_type = (
      jax.ShapeDtypeStruct((B, C), jnp.float32),
      jax.ShapeDtypeStruct((B, C), jnp.int32),
      jax.ShapeDtypeStruct((B, C), jnp.int32),
      jax.ShapeDtypeStruct((B, C), jnp.int32),
      jax.ShapeDtypeStruct((B, C), jnp.int32),
  )
  scratch = [
      pltpu.VMEM((C,), jnp.float32),
      pltpu.VMEM((C,), jnp.float32),
      pltpu.VMEM((C,), jnp.int32),
      pltpu.VMEM((C,), jnp.int32),
      pltpu.VMEM((C,), jnp.int32),
      pltpu.VMEM((C,), jnp.int32),
      pltpu.VMEM((NSPAN,), jnp.int32),
      pltpu.VMEM((NSPAN,), jnp.int32),
      pltpu.VMEM((C,), jnp.int32),
      pltpu.VMEM((C,), jnp.int32),
      pltpu.VMEM((C,), jnp.int32),
      pltpu.VMEM((C,), jnp.float32),
      pltpu.VMEM((C,), jnp.int32),
      pltpu.VMEM((256,), jnp.int32),
      pltpu.VMEM((256,), jnp.int32),
      pltpu.VMEM((512,), jnp.int32),
      pltpu.VMEM((512,), jnp.int32),
  ]
  f = pl.kernel(_nms_body, out_type=out_type, mesh=mesh,
                scratch_types=scratch,
                compiler_params=pltpu.CompilerParams(
                    needs_layout_passes=False))
  ks, kp, ss, es, lb = f(sc, st, en)
  keep = kp.astype(bool)
  sp = jnp.stack([ss, es], axis=-1)
  return ks, keep, sp, lb


# speedup vs baseline: 401.3332x; 1.1626x over previous
import jax
import jax.numpy as jnp
from jax import lax
from jax.experimental import pallas as pl
from jax.experimental.pallas import tpu as pltpu
from jax.experimental.pallas import tpu_sc as plsc

B = 4
C = 8000
NV = C // 16
NQ = 4
NVQ = NV // NQ
NBIN = 1024
NBV = NBIN // 16
NSPAN = 1000
THR = 0.5
KEY_BIAS = 0x7FFFFFFF


def _nms_body(sc_hbm, st_hbm, en_hbm,
              ks_hbm, kp_hbm, ss_hbm, es_hbm, lb_hbm,
              sc_v, scs_v, key_a, idx_a, key_b, idx_b,
              st_in, en_in, ss_v, es_v, lb_v, ks_v, kp_v,
              h0, h1, h2, h3, p0, p1, p2, p3, cov, pref):
  wid = lax.axis_index("s") * 2 + lax.axis_index("c")
  lane = lax.iota(jnp.int32, 16)
  hq = [h0, h1, h2, h3]
  pq = [p0, p1, p2, p3]

  @pl.when(wid < B)
  def _():
    b = wid
    pltpu.sync_copy(sc_hbm.at[b], sc_v)
    pltpu.sync_copy(st_hbm.at[b], st_in)
    pltpu.sync_copy(en_hbm.at[b], en_in)

    for p, (sk, si, dk, di) in enumerate(
        [(None, None, key_a, idx_a),
         (key_a, idx_a, key_b, idx_b),
         (key_b, idx_b, key_a, idx_a)]):
      shift = 10 * p

      def zero_h(j, _):
        for q in range(NQ):
          hq[q][pl.ds(j * 16, 16)] = jnp.zeros((16,), jnp.int32)
        return 0
      lax.fori_loop(0, NBV, zero_h, 0)

      def load_kv(q, i, sk=sk, si=si):
        if sk is None:
          off = (q * NVQ + i) * 16
          s = sc_v[pl.ds(off, 16)]
          k = KEY_BIAS - lax.bitcast_convert_type(s, jnp.int32)
          ii = lane + off
        else:
          off = (q * NVQ + i) * 16
          k = sk[pl.ds(off, 16)]
          ii = si[pl.ds(off, 16)]
        return k, ii

      def hist_i(i, _, shift=shift, load_kv=load_kv):
        for q in range(NQ):
          k, _ii = load_kv(q, i)
          d = lax.shift_right_logical(k, shift) & (NBIN - 1)
          cnt, last = plsc.scan_count(d)
          plsc.addupdate_scatter(hq[q], [d], cnt, mask=last)
        return 0
      lax.fori_loop(0, NVQ, hist_i, 0)

      def scan_h(j, carry):
        hv = [hq[q][pl.ds(j * 16, 16)] for q in range(NQ)]
        tot = hv[0] + hv[1] + hv[2] + hv[3]
        c = plsc.cumsum(tot)
        excl = c - tot + carry
        acc = excl
        for q in range(NQ):
          pq[q][pl.ds(j * 16, 16)] = acc
          if q < NQ - 1:
            acc = acc + hv[q]
        return carry + jnp.max(c)
      lax.fori_loop(0, NBV, scan_h, jnp.int32(0))

      def perm_i(i, _, shift=shift, load_kv=load_kv, dk=dk, di=di):
        for q in range(NQ):
          k, ii = load_kv(q, i)
          d = lax.shift_right_logical(k, shift) & (NBIN - 1)
          cnt, last = plsc.scan_count(d)
          base = plsc.load_gather(pq[q], [d])
          tgt = base + cnt - 1
          plsc.store_scatter(dk, [tgt], k)
          plsc.store_scatter(di, [tgt], ii)
          plsc.addupdate_scatter(pq[q], [d], cnt, mask=last)
        return 0
      lax.fori_loop(0, NVQ, perm_i, 0)

    def post_i(i, _):
      ii = idx_a[pl.ds(i * 16, 16)]
      k = key_a[pl.ds(i * 16, 16)]
      sidx = lax.shift_right_logical(ii, 3)
      ss_v[pl.ds(i * 16, 16)] = plsc.load_gather(st_in, [sidx])
      es_v[pl.ds(i * 16, 16)] = plsc.load_gather(en_in, [sidx])
      lb_v[pl.ds(i * 16, 16)] = ii & 7
      scs_v[pl.ds(i * 16, 16)] = lax.bitcast_convert_type(KEY_BIAS - k,
                                                          jnp.float32)
      ks_v[pl.ds(i * 16, 16)] = jnp.zeros((16,), jnp.float32)
      kp_v[pl.ds(i * 16, 16)] = jnp.zeros((16,), jnp.int32)
      return 0
    lax.fori_loop(0, NV, post_i, 0)

    def zero_cp(j, _):
      cov[pl.ds(j * 16, 16)] = jnp.zeros((16,), jnp.int32)
      pref[pl.ds(j * 16, 16)] = jnp.zeros((16,), jnp.int32)
      return 0
    lax.fori_loop(0, 32, zero_cp, 0)

    def g_cond(cr):
      g, cont = cr
      return jnp.logical_and(g < NV, cont)

    def g_body(cr):
      g, _ = cr
      off = g * 16
      scv = scs_v[pl.ds(off, 16)]
      anyv = jnp.max(scv) > THR

      @pl.when(anyv)
      def _():
        valid = scv > THR
        st = ss_v[pl.ds(off, 16)]
        en = es_v[pl.ds(off, 16)]

        def i_cond(ic):
          return ic[1]

        def i_body(ic):
          prev_k, _, kvec = ic
          pfs = plsc.load_gather(pref, [st])
          pfe = plsc.load_gather(pref, [en])
          cvs = plsc.load_gather(cov, [st])
          conf = (pfe - pfs + cvs) > 0
          cand = jnp.logical_and(
              jnp.logical_and(valid, jnp.logical_not(conf)), lane > prev_k)
          npop = jnp.max(plsc.all_reduce_population_count(cand))
          has = npop > 0
          kidx = jnp.max(plsc.all_reduce_ffs(cand))

          @pl.when(has)
          def _():
            onehot = lane == kidx
            s_k = jnp.max(jnp.where(onehot, st, jnp.int32(-1)))
            e_k = jnp.max(jnp.where(onehot, en, jnp.int32(-1)))

            def upd(j, carry):
              gpos = lane + j * 16
              cvj = cov[pl.ds(j * 16, 16)]
              m = jnp.logical_and(gpos >= s_k, gpos <= e_k)
              cvj = jnp.where(m, jnp.int32(1), cvj)
              cov[pl.ds(j * 16, 16)] = cvj
              cs = plsc.cumsum(cvj)
              pref[pl.ds(j * 16, 16)] = cs + carry
              return carry + jnp.max(cs)
            lax.fori_loop(0, 32, upd, jnp.int32(0))

          kvec2 = jnp.where(jnp.logical_and(has, lane == kidx),
                            jnp.int32(1), kvec)
          prev2 = jnp.where(has, kidx, prev_k)
          return (prev2, has, kvec2)

        _, _, kfin = lax.while_loop(
            i_cond, i_body,
            (jnp.int32(-1), True, jnp.zeros((16,), jnp.int32)))
        kp_v[pl.ds(off, 16)] = kfin
        ks_v[pl.ds(off, 16)] = scv * kfin.astype(jnp.float32)

      return (g + 1, anyv)

    lax.while_loop(g_cond, g_body, (jnp.int32(0), True))

    pltpu.sync_copy(ks_v, ks_hbm.at[b])
    pltpu.sync_copy(kp_v, kp_hbm.at[b])
    pltpu.sync_copy(ss_v, ss_hbm.at[b])
    pltpu.sync_copy(es_v, es_hbm.at[b])
    pltpu.sync_copy(lb_v, lb_hbm.at[b])


def kernel(probs_batch, span_indices_batch):
  sc = probs_batch.reshape(B, C)
  st = span_indices_batch[..., 0]
  en = span_indices_batch[..., 1]

  mesh = plsc.VectorSubcoreMesh(core_axis_name="c", subcore_axis_name="s")
  out_type = (
      jax.ShapeDtypeStruct((B, C), jnp.float32),
      jax.ShapeDtypeStruct((B, C), jnp.int32),
      jax.ShapeDtypeStruct((B, C), jnp.int32),
      jax.ShapeDtypeStruct((B, C), jnp.int32),
      jax.ShapeDtypeStruct((B, C), jnp.int32),
  )
  scratch = [
      pltpu.VMEM((C,), jnp.float32),
      pltpu.VMEM((C,), jnp.float32),
      pltpu.VMEM((C,), jnp.int32),
      pltpu.VMEM((C,), jnp.int32),
      pltpu.VMEM((C,), jnp.int32),
      pltpu.VMEM((C,), jnp.int32),
      pltpu.VMEM((NSPAN,), jnp.int32),
      pltpu.VMEM((NSPAN,), jnp.int32),
      pltpu.VMEM((C,), jnp.int32),
      pltpu.VMEM((C,), jnp.int32),
      pltpu.VMEM((C,), jnp.int32),
      pltpu.VMEM((C,), jnp.float32),
      pltpu.VMEM((C,), jnp.int32),
  ] + [pltpu.VMEM((NBIN,), jnp.int32) for _ in range(2 * NQ)] + [
      pltpu.VMEM((512,), jnp.int32),
      pltpu.VMEM((512,), jnp.int32),
  ]
  f = pl.kernel(_nms_body, out_type=out_type, mesh=mesh,
                scratch_types=scratch,
                compiler_params=pltpu.CompilerParams(
                    needs_layout_passes=False))
  ks, kp, ss, es, lb = f(sc, st, en)
  keep = kp.astype(bool)
  sp = jnp.stack([ss, es], axis=-1)
  return ks, keep, sp, lb


# speedup vs baseline: 412.0162x; 1.0266x over previous
import jax
import jax.numpy as jnp
from jax import lax
from jax.experimental import pallas as pl
from jax.experimental.pallas import tpu as pltpu
from jax.experimental.pallas import tpu_sc as plsc

B = 4
C = 8000
NV = C // 16
NQ = 4
NVQ = NV // NQ
NBIN = 1024
NBV = NBIN // 16
NSPAN = 1000
THR = 0.5
KEY_BIAS = 0x7FFFFFFF


def _nms_body(sc_hbm, st_hbm, en_hbm,
              ks_hbm, kp_hbm, ss_hbm, es_hbm, lb_hbm,
              sc_v, scs_v, key_a, idx_a, key_b, idx_b,
              st_in, en_in, ss_v, es_v, lb_v, ks_v, kp_v,
              h0, h1, h2, h3, p0, p1, p2, p3, cov, prefE, pref2, sem):
  wid = lax.axis_index("s") * 2 + lax.axis_index("c")
  lane = lax.iota(jnp.int32, 16)
  hq = [h0, h1, h2, h3]
  pq = [p0, p1, p2, p3]

  @pl.when(wid < B)
  def _():
    b = wid
    pltpu.sync_copy(sc_hbm.at[b], sc_v)
    pltpu.sync_copy(st_hbm.at[b], st_in)
    pltpu.sync_copy(en_hbm.at[b], en_in)

    for p, (sk, si, dk, di) in enumerate(
        [(None, None, key_a, idx_a),
         (key_a, idx_a, key_b, idx_b),
         (key_b, idx_b, key_a, idx_a)]):
      shift = 10 * p

      def zero_h(j, _):
        for q in range(NQ):
          hq[q][pl.ds(j * 16, 16)] = jnp.zeros((16,), jnp.int32)
        return 0
      lax.fori_loop(0, NBV, zero_h, 0)

      def load_kv(q, i, sk=sk, si=si):
        if sk is None:
          off = (q * NVQ + i) * 16
          s = sc_v[pl.ds(off, 16)]
          k = KEY_BIAS - lax.bitcast_convert_type(s, jnp.int32)
          ii = lane + off
        else:
          off = (q * NVQ + i) * 16
          k = sk[pl.ds(off, 16)]
          ii = si[pl.ds(off, 16)]
        return k, ii

      def hist_i(i, _, shift=shift, load_kv=load_kv):
        for q in range(NQ):
          k, _ii = load_kv(q, i)
          d = lax.shift_right_logical(k, shift) & (NBIN - 1)
          cnt, last = plsc.scan_count(d)
          plsc.addupdate_scatter(hq[q], [d], cnt, mask=last)
        return 0
      lax.fori_loop(0, NVQ, hist_i, 0)

      def scan_h(j, carry):
        hv = [hq[q][pl.ds(j * 16, 16)] for q in range(NQ)]
        tot = hv[0] + hv[1] + hv[2] + hv[3]
        c = plsc.cumsum(tot)
        excl = c - tot + carry
        acc = excl
        for q in range(NQ):
          pq[q][pl.ds(j * 16, 16)] = acc
          if q < NQ - 1:
            acc = acc + hv[q]
        return carry + jnp.max(c)
      lax.fori_loop(0, NBV, scan_h, jnp.int32(0))

      def perm_i(i, _, p=p, shift=shift, load_kv=load_kv, dk=dk, di=di):
        for q in range(NQ):
          k, ii = load_kv(q, i)
          d = lax.shift_right_logical(k, shift) & (NBIN - 1)
          cnt, last = plsc.scan_count(d)
          base = plsc.load_gather(pq[q], [d])
          tgt = base + cnt - 1
          if p < 2:
            plsc.store_scatter(dk, [tgt], k)
          plsc.store_scatter(di, [tgt], ii)
          plsc.addupdate_scatter(pq[q], [d], cnt, mask=last)
        return 0
      lax.fori_loop(0, NVQ, perm_i, 0)

    def post_i(i, _):
      ii = idx_a[pl.ds(i * 16, 16)]
      sidx = lax.shift_right_logical(ii, 3)
      ss_v[pl.ds(i * 16, 16)] = plsc.load_gather(st_in, [sidx])
      es_v[pl.ds(i * 16, 16)] = plsc.load_gather(en_in, [sidx])
      lb_v[pl.ds(i * 16, 16)] = ii & 7
      scs_v[pl.ds(i * 16, 16)] = plsc.load_gather(sc_v, [ii])
      ks_v[pl.ds(i * 16, 16)] = jnp.zeros((16,), jnp.float32)
      kp_v[pl.ds(i * 16, 16)] = jnp.zeros((16,), jnp.int32)
      return 0
    lax.fori_loop(0, NV, post_i, 0)

    def zero_cp(j, _):
      cov[pl.ds(j * 16, 16)] = jnp.zeros((16,), jnp.int32)
      prefE[pl.ds(j * 16, 16)] = jnp.zeros((16,), jnp.int32)
      pref2[pl.ds(j * 16, 16)] = jnp.zeros((16,), jnp.int32)
      return 0
    lax.fori_loop(0, 32, zero_cp, 0)

    def bs_body(_, cr):
      lo, hi = cr
      mid = lax.div(lo + hi, 2)
      v = scs_v[pl.ds(mid, 16)]
      gt = v[0] > THR
      return (jnp.where(gt, mid + 1, lo), jnp.where(gt, hi, mid))
    count, _ = lax.fori_loop(0, 13, bs_body, (jnp.int32(0), jnp.int32(C)))
    ngroups = lax.div(count + 15, 16)

    def g_body(g, _):
      off = g * 16
      scv = scs_v[pl.ds(off, 16)]
      valid = (off + lane) < count
      st = ss_v[pl.ds(off, 16)]
      en = es_v[pl.ds(off, 16)]

      def i_cond(ic):
        return ic[1]

      def i_body(ic):
        prev_k, _, kvec = ic
        pfs = plsc.load_gather(pref2, [st])
        pfe = plsc.load_gather(prefE, [en])
        conf = (pfe - pfs) > 0
        cand = jnp.logical_and(
            jnp.logical_and(valid, jnp.logical_not(conf)), lane > prev_k)
        kidx = jnp.max(plsc.all_reduce_ffs(cand))
        has = jnp.logical_and(kidx >= 0, kidx < 16)

        @pl.when(has)
        def _():
          onehot = lane == kidx
          s_k = jnp.max(jnp.where(onehot, st, jnp.int32(-1)))
          e_k = jnp.max(jnp.where(onehot, en, jnp.int32(-1)))

          def upd(j, carry):
            gpos = lane + j * 16
            cvj = cov[pl.ds(j * 16, 16)]
            m = jnp.logical_and(gpos >= s_k, gpos <= e_k)
            cvj = jnp.where(m, jnp.int32(1), cvj)
            cov[pl.ds(j * 16, 16)] = cvj
            cs = plsc.cumsum(cvj)
            prefE[pl.ds(j * 16, 16)] = cs + carry
            pref2[pl.ds(j * 16, 16)] = cs - cvj + carry
            return carry + jnp.max(cs)
          lax.fori_loop(0, 32, upd, jnp.int32(0))

        kvec2 = jnp.where(jnp.logical_and(has, lane == kidx),
                          jnp.int32(1), kvec)
        prev2 = jnp.where(has, kidx, prev_k)
        return (prev2, has, kvec2)

      _, _, kfin = lax.while_loop(
          i_cond, i_body,
          (jnp.int32(-1), True, jnp.zeros((16,), jnp.int32)))
      kp_v[pl.ds(off, 16)] = kfin
      ks_v[pl.ds(off, 16)] = scv * kfin.astype(jnp.float32)
      return 0

    lax.fori_loop(0, ngroups, g_body, 0)

    d1 = pltpu.async_copy(ks_v, ks_hbm.at[b], sem)
    d2 = pltpu.async_copy(kp_v, kp_hbm.at[b], sem)
    d3 = pltpu.async_copy(ss_v, ss_hbm.at[b], sem)
    d4 = pltpu.async_copy(es_v, es_hbm.at[b], sem)
    d5 = pltpu.async_copy(lb_v, lb_hbm.at[b], sem)
    d1.wait()
    d2.wait()
    d3.wait()
    d4.wait()
    d5.wait()


def kernel(probs_batch, span_indices_batch):
  sc = probs_batch.reshape(B, C)
  st = span_indices_batch[..., 0]
  en = span_indices_batch[..., 1]

  mesh = plsc.VectorSubcoreMesh(core_axis_name="c", subcore_axis_name="s")
  out_type = (
      jax.ShapeDtypeStruct((B, C), jnp.float32),
      jax.ShapeDtypeStruct((B, C), jnp.int32),
      jax.ShapeDtypeStruct((B, C), jnp.int32),
      jax.ShapeDtypeStruct((B, C), jnp.int32),
      jax.ShapeDtypeStruct((B, C), jnp.int32),
  )
  scratch = [
      pltpu.VMEM((C,), jnp.float32),
      pltpu.VMEM((C + 16,), jnp.float32),
      pltpu.VMEM((C,), jnp.int32),
      pltpu.VMEM((C,), jnp.int32),
      pltpu.VMEM((C,), jnp.int32),
      pltpu.VMEM((C,), jnp.int32),
      pltpu.VMEM((NSPAN,), jnp.int32),
      pltpu.VMEM((NSPAN,), jnp.int32),
      pltpu.VMEM((C,), jnp.int32),
      pltpu.VMEM((C,), jnp.int32),
      pltpu.VMEM((C,), jnp.int32),
      pltpu.VMEM((C,), jnp.float32),
      pltpu.VMEM((C,), jnp.int32),
  ] + [pltpu.VMEM((NBIN,), jnp.int32) for _ in range(2 * NQ)] + [
      pltpu.VMEM((512,), jnp.int32),
      pltpu.VMEM((512,), jnp.int32),
      pltpu.VMEM((512,), jnp.int32),
      pltpu.SemaphoreType.DMA,
  ]
  f = pl.kernel(_nms_body, out_type=out_type, mesh=mesh,
                scratch_types=scratch,
                compiler_params=pltpu.CompilerParams(
                    needs_layout_passes=False))
  ks, kp, ss, es, lb = f(sc, st, en)
  keep = kp.astype(bool)
  sp = jnp.stack([ss, es], axis=-1)
  return ks, keep, sp, lb


# speedup vs baseline: 418.0789x; 1.0147x over previous
import jax
import jax.numpy as jnp
from jax import lax
from jax.experimental import pallas as pl
from jax.experimental.pallas import tpu as pltpu
from jax.experimental.pallas import tpu_sc as plsc

B = 4
C = 8000
NV = C // 16
NQ = 4
NVQ = NV // NQ
NBIN = 1024
NBV = NBIN // 16
NSPAN = 1000
THR = 0.5
KEY_BIAS = 0x7FFFFFFF


def _nms_body(sc_hbm, st_hbm, en_hbm,
              ks_hbm, kp_hbm, ss_hbm, es_hbm, lb_hbm,
              sc_v, scs_v, key_a, idx_a, key_b, idx_b,
              st_in, en_in, ss_v, es_v, lb_v, ks_v, kp_v,
              h0, h1, h2, h3, p0, p1, p2, p3, cov, prefE, pref2, sem):
  wid = lax.axis_index("s") * 2 + lax.axis_index("c")
  lane = lax.iota(jnp.int32, 16)
  hq = [h0, h1, h2, h3]
  pq = [p0, p1, p2, p3]

  @pl.when(wid < B)
  def _():
    b = wid
    pltpu.sync_copy(sc_hbm.at[b], sc_v)
    pltpu.sync_copy(st_hbm.at[b], st_in)
    pltpu.sync_copy(en_hbm.at[b], en_in)

    for p, (sk, si, dk, di) in enumerate(
        [(None, None, key_a, idx_a),
         (key_a, idx_a, key_b, idx_b),
         (key_b, idx_b, key_a, idx_a)]):
      shift = 10 * p

      def zero_h(j, _):
        for q in range(NQ):
          hq[q][pl.ds(j * 16, 16)] = jnp.zeros((16,), jnp.int32)
        return 0
      lax.fori_loop(0, NBV, zero_h, 0)

      def load_kv(q, i, sk=sk, si=si):
        if sk is None:
          off = (q * NVQ + i) * 16
          s = sc_v[pl.ds(off, 16)]
          k = KEY_BIAS - lax.bitcast_convert_type(s, jnp.int32)
          ii = lane + off
        else:
          off = (q * NVQ + i) * 16
          k = sk[pl.ds(off, 16)]
          ii = si[pl.ds(off, 16)]
        return k, ii

      def hist_i(i, _, shift=shift, load_kv=load_kv):
        for q in range(NQ):
          k, _ii = load_kv(q, i)
          d = lax.shift_right_logical(k, shift) & (NBIN - 1)
          cnt, last = plsc.scan_count(d)
          plsc.addupdate_scatter(hq[q], [d], cnt, mask=last)
        return 0
      lax.fori_loop(0, NVQ, hist_i, 0)

      def scan_h(j, carry):
        hv = [hq[q][pl.ds(j * 16, 16)] for q in range(NQ)]
        tot = hv[0] + hv[1] + hv[2] + hv[3]
        c = plsc.cumsum(tot)
        excl = c - tot + carry
        acc = excl
        for q in range(NQ):
          pq[q][pl.ds(j * 16, 16)] = acc
          if q < NQ - 1:
            acc = acc + hv[q]
        return carry + jnp.max(c)
      lax.fori_loop(0, NBV, scan_h, jnp.int32(0))

      def perm_i(i, _, p=p, shift=shift, load_kv=load_kv, dk=dk, di=di):
        for q in range(NQ):
          k, ii = load_kv(q, i)
          d = lax.shift_right_logical(k, shift) & (NBIN - 1)
          cnt, last = plsc.scan_count(d)
          base = plsc.load_gather(pq[q], [d])
          tgt = base + cnt - 1
          if p < 2:
            plsc.store_scatter(dk, [tgt], k)
          plsc.store_scatter(di, [tgt], ii)
          plsc.addupdate_scatter(pq[q], [d], cnt, mask=last)
        return 0
      lax.fori_loop(0, NVQ, perm_i, 0)

    def post_i(i, _):
      ii = idx_a[pl.ds(i * 16, 16)]
      sidx = lax.shift_right_logical(ii, 3)
      ss_v[pl.ds(i * 16, 16)] = plsc.load_gather(st_in, [sidx])
      es_v[pl.ds(i * 16, 16)] = plsc.load_gather(en_in, [sidx])
      lb_v[pl.ds(i * 16, 16)] = ii & 7
      scs_v[pl.ds(i * 16, 16)] = plsc.load_gather(sc_v, [ii])
      ks_v[pl.ds(i * 16, 16)] = jnp.zeros((16,), jnp.float32)
      kp_v[pl.ds(i * 16, 16)] = jnp.zeros((16,), jnp.int32)
      return 0
    lax.fori_loop(0, NV, post_i, 0)

    def zero_cp(j, _):
      cov[pl.ds(j * 16, 16)] = jnp.zeros((16,), jnp.int32)
      prefE[pl.ds(j * 16, 16)] = jnp.zeros((16,), jnp.int32)
      pref2[pl.ds(j * 16, 16)] = jnp.zeros((16,), jnp.int32)
      return 0
    lax.fori_loop(0, 32, zero_cp, 0)

    def bs_body(_, cr):
      lo, hi = cr
      mid = lax.div(lo + hi, 2)
      v = scs_v[pl.ds(mid, 16)]
      gt = v[0] > THR
      return (jnp.where(gt, mid + 1, lo), jnp.where(gt, hi, mid))
    count, _ = lax.fori_loop(0, 13, bs_body, (jnp.int32(0), jnp.int32(C)))
    ngroups = lax.div(count + 15, 16)

    def g_body(g, _):
      off = g * 16
      valid = (off + lane) < count
      st = ss_v[pl.ds(off, 16)]
      en = es_v[pl.ds(off, 16)]

      def i_cond(ic):
        return ic[1]

      def i_body(ic):
        prev_k, _ = ic
        pfs = plsc.load_gather(pref2, [st])
        pfe = plsc.load_gather(prefE, [en])
        conf = (pfe - pfs) > 0
        cand = jnp.logical_and(
            jnp.logical_and(valid, jnp.logical_not(conf)), lane > prev_k)
        kidx = plsc.all_reduce_ffs(cand)[0]
        has = jnp.logical_and(kidx >= 0, kidx < 16)

        @pl.when(has)
        def _():
          onehot = lane == kidx
          s_k = jnp.max(jnp.where(onehot, st, jnp.int32(-1)))
          e_k = jnp.max(jnp.where(onehot, en, jnp.int32(-1)))

          def upd(j, carry):
            gpos = lane + j * 16
            cvj = cov[pl.ds(j * 16, 16)]
            m = jnp.logical_and(gpos >= s_k, gpos <= e_k)
            cvj = jnp.where(m, jnp.int32(1), cvj)
            cov[pl.ds(j * 16, 16)] = cvj
            cs = plsc.cumsum(cvj)
            prefE[pl.ds(j * 16, 16)] = cs + carry
            pref2[pl.ds(j * 16, 16)] = cs - cvj + carry
            return carry + jnp.max(cs)
          lax.fori_loop(0, 32, upd, jnp.int32(0))

          kv = kp_v[pl.ds(off, 16)] | jnp.where(onehot, jnp.int32(1),
                                                jnp.int32(0))
          kp_v[pl.ds(off, 16)] = kv
          scv = scs_v[pl.ds(off, 16)]
          ks_v[pl.ds(off, 16)] = scv * kv.astype(jnp.float32)

        prev2 = jnp.where(has, kidx, prev_k)
        return (prev2, has)

      lax.while_loop(i_cond, i_body, (jnp.int32(-1), True))
      return 0

    lax.fori_loop(0, ngroups, g_body, 0)

    d1 = pltpu.async_copy(ks_v, ks_hbm.at[b], sem)
    d2 = pltpu.async_copy(kp_v, kp_hbm.at[b], sem)
    d3 = pltpu.async_copy(ss_v, ss_hbm.at[b], sem)
    d4 = pltpu.async_copy(es_v, es_hbm.at[b], sem)
    d5 = pltpu.async_copy(lb_v, lb_hbm.at[b], sem)
    d1.wait()
    d2.wait()
    d3.wait()
    d4.wait()
    d5.wait()


def kernel(probs_batch, span_indices_batch):
  sc = probs_batch.reshape(B, C)
  st = span_indices_batch[..., 0]
  en = span_indices_batch[..., 1]

  mesh = plsc.VectorSubcoreMesh(core_axis_name="c", subcore_axis_name="s")
  out_type = (
      jax.ShapeDtypeStruct((B, C), jnp.float32),
      jax.ShapeDtypeStruct((B, C), jnp.int32),
      jax.ShapeDtypeStruct((B, C), jnp.int32),
      jax.ShapeDtypeStruct((B, C), jnp.int32),
      jax.ShapeDtypeStruct((B, C), jnp.int32),
  )
  scratch = [
      pltpu.VMEM((C,), jnp.float32),
      pltpu.VMEM((C + 16,), jnp.float32),
      pltpu.VMEM((C,), jnp.int32),
      pltpu.VMEM((C,), jnp.int32),
      pltpu.VMEM((C,), jnp.int32),
      pltpu.VMEM((C,), jnp.int32),
      pltpu.VMEM((NSPAN,), jnp.int32),
      pltpu.VMEM((NSPAN,), jnp.int32),
      pltpu.VMEM((C,), jnp.int32),
      pltpu.VMEM((C,), jnp.int32),
      pltpu.VMEM((C,), jnp.int32),
      pltpu.VMEM((C,), jnp.float32),
      pltpu.VMEM((C,), jnp.int32),
  ] + [pltpu.VMEM((NBIN,), jnp.int32) for _ in range(2 * NQ)] + [
      pltpu.VMEM((512,), jnp.int32),
      pltpu.VMEM((512,), jnp.int32),
      pltpu.VMEM((512,), jnp.int32),
      pltpu.SemaphoreType.DMA,
  ]
  f = pl.kernel(_nms_body, out_type=out_type, mesh=mesh,
                scratch_types=scratch,
                compiler_params=pltpu.CompilerParams(
                    needs_layout_passes=False))
  ks, kp, ss, es, lb = f(sc, st, en)
  keep = kp.astype(bool)
  sp = jnp.stack([ss, es], axis=-1)
  return ks, keep, sp, lb


# speedup vs baseline: 437.6419x; 1.0468x over previous
import jax
import jax.numpy as jnp
from jax import lax
from jax.experimental import pallas as pl
from jax.experimental.pallas import tpu as pltpu
from jax.experimental.pallas import tpu_sc as plsc

B = 4
C = 8000
NV = C // 16
NQ = 4
NVQ = NV // NQ
NBIN = 1024
NBV = NBIN // 16
NSPAN = 1000
THR = 0.5
KEY_BIAS = 0x7FFFFFFF


def _nms_body(sc_hbm, st_hbm, en_hbm,
              ks_hbm, kp_hbm, ss_hbm, es_hbm, lb_hbm,
              sc_v, scs_v, key_a, idx_a, key_b, idx_b,
              st_in, en_in, ss_v, es_v, lb_v, ks_v, kp_v,
              h0, h1, h2, h3, p0, p1, p2, p3, cov, prefE, pref2, sem):
  wid = lax.axis_index("s") * 2 + lax.axis_index("c")
  lane = lax.iota(jnp.int32, 16)
  hq = [h0, h1, h2, h3]
  pq = [p0, p1, p2, p3]

  @pl.when(wid < B)
  def _():
    b = wid
    pltpu.sync_copy(sc_hbm.at[b], sc_v)
    pltpu.sync_copy(st_hbm.at[b], st_in)
    pltpu.sync_copy(en_hbm.at[b], en_in)

    for p, (sk, si, dk, di) in enumerate(
        [(None, None, key_a, idx_a),
         (key_a, idx_a, key_b, idx_b),
         (key_b, idx_b, key_a, idx_a)]):
      shift = 10 * p

      @plsc.parallel_loop(0, NBIN, 16, unroll=4)
      def zero_h(j):
        for q in range(NQ):
          hq[q][pl.ds(j, 16)] = jnp.zeros((16,), jnp.int32)

      def load_kv(q, i, sk=sk, si=si):
        if sk is None:
          off = (q * NVQ + i) * 16
          s = sc_v[pl.ds(off, 16)]
          k = KEY_BIAS - lax.bitcast_convert_type(s, jnp.int32)
          ii = lane + off
        else:
          off = (q * NVQ + i) * 16
          k = sk[pl.ds(off, 16)]
          ii = si[pl.ds(off, 16)]
        return k, ii

      def hist_i(i, _, shift=shift, load_kv=load_kv):
        for q in range(NQ):
          k, _ii = load_kv(q, i)
          d = lax.shift_right_logical(k, shift) & (NBIN - 1)
          cnt, last = plsc.scan_count(d)
          plsc.addupdate_scatter(hq[q], [d], cnt, mask=last)
        return 0
      lax.fori_loop(0, NVQ, hist_i, 0)

      def scan_h(j, carry):
        hv = [hq[q][pl.ds(j * 16, 16)] for q in range(NQ)]
        tot = hv[0] + hv[1] + hv[2] + hv[3]
        c = plsc.cumsum(tot)
        excl = c - tot + carry
        acc = excl
        for q in range(NQ):
          pq[q][pl.ds(j * 16, 16)] = acc
          if q < NQ - 1:
            acc = acc + hv[q]
        return carry + jnp.max(c)
      lax.fori_loop(0, NBV, scan_h, jnp.int32(0))

      def perm_i(i, _, p=p, shift=shift, load_kv=load_kv, dk=dk, di=di):
        for q in range(NQ):
          k, ii = load_kv(q, i)
          d = lax.shift_right_logical(k, shift) & (NBIN - 1)
          cnt, last = plsc.scan_count(d)
          base = plsc.load_gather(pq[q], [d])
          tgt = base + cnt - 1
          if p < 2:
            plsc.store_scatter(dk, [tgt], k)
          plsc.store_scatter(di, [tgt], ii)
          plsc.addupdate_scatter(pq[q], [d], cnt, mask=last)
        return 0
      lax.fori_loop(0, NVQ, perm_i, 0)

    @plsc.parallel_loop(0, C, 16, unroll=4)
    def post_i(i):
      ii = idx_a[pl.ds(i, 16)]
      sidx = lax.shift_right_logical(ii, 3)
      ss_v[pl.ds(i, 16)] = plsc.load_gather(st_in, [sidx])
      es_v[pl.ds(i, 16)] = plsc.load_gather(en_in, [sidx])
      lb_v[pl.ds(i, 16)] = ii & 7
      scs_v[pl.ds(i, 16)] = plsc.load_gather(sc_v, [ii])
      ks_v[pl.ds(i, 16)] = jnp.zeros((16,), jnp.float32)
      kp_v[pl.ds(i, 16)] = jnp.zeros((16,), jnp.int32)

    @plsc.parallel_loop(0, 512, 16, unroll=4)
    def zero_cp(j):
      cov[pl.ds(j, 16)] = jnp.zeros((16,), jnp.int32)
      prefE[pl.ds(j, 16)] = jnp.zeros((16,), jnp.int32)
      pref2[pl.ds(j, 16)] = jnp.zeros((16,), jnp.int32)

    def bs_body(_, cr):
      lo, hi = cr
      mid = lax.div(lo + hi, 2)
      v = scs_v[pl.ds(mid, 16)]
      gt = v[0] > THR
      return (jnp.where(gt, mid + 1, lo), jnp.where(gt, hi, mid))
    count, _ = lax.fori_loop(0, 13, bs_body, (jnp.int32(0), jnp.int32(C)))
    ngroups = lax.div(count + 15, 16)

    def g_body(g, _):
      off = g * 16
      valid = (off + lane) < count
      st = ss_v[pl.ds(off, 16)]
      en = es_v[pl.ds(off, 16)]

      def i_cond(ic):
        return ic[1]

      def i_body(ic):
        prev_k, _ = ic
        pfs = plsc.load_gather(pref2, [st])
        pfe = plsc.load_gather(prefE, [en])
        conf = (pfe - pfs) > 0
        cand = jnp.logical_and(
            jnp.logical_and(valid, jnp.logical_not(conf)), lane > prev_k)
        kidx = plsc.all_reduce_ffs(cand)[0]
        has = jnp.logical_and(kidx >= 0, kidx < 16)

        @pl.when(has)
        def _():
          onehot = lane == kidx
          s_k = jnp.max(jnp.where(onehot, st, jnp.int32(-1)))
          e_k = jnp.max(jnp.where(onehot, en, jnp.int32(-1)))

          def upd(j, carry):
            gpos = lane + j * 16
            cvj = cov[pl.ds(j * 16, 16)]
            m = jnp.logical_and(gpos >= s_k, gpos <= e_k)
            cvj = jnp.where(m, jnp.int32(1), cvj)
            cov[pl.ds(j * 16, 16)] = cvj
            cs = plsc.cumsum(cvj)
            prefE[pl.ds(j * 16, 16)] = cs + carry
            pref2[pl.ds(j * 16, 16)] = cs - cvj + carry
            return carry + jnp.max(cs)
          lax.fori_loop(0, 32, upd, jnp.int32(0))

          kv = kp_v[pl.ds(off, 16)] | jnp.where(onehot, jnp.int32(1),
                                                jnp.int32(0))
          kp_v[pl.ds(off, 16)] = kv
          scv = scs_v[pl.ds(off, 16)]
          ks_v[pl.ds(off, 16)] = scv * kv.astype(jnp.float32)

        prev2 = jnp.where(has, kidx, prev_k)
        return (prev2, has)

      lax.while_loop(i_cond, i_body, (jnp.int32(-1), True))
      return 0

    lax.fori_loop(0, ngroups, g_body, 0)

    d1 = pltpu.async_copy(ks_v, ks_hbm.at[b], sem)
    d2 = pltpu.async_copy(kp_v, kp_hbm.at[b], sem)
    d3 = pltpu.async_copy(ss_v, ss_hbm.at[b], sem)
    d4 = pltpu.async_copy(es_v, es_hbm.at[b], sem)
    d5 = pltpu.async_copy(lb_v, lb_hbm.at[b], sem)
    d1.wait()
    d2.wait()
    d3.wait()
    d4.wait()
    d5.wait()


def kernel(probs_batch, span_indices_batch):
  sc = probs_batch.reshape(B, C)
  st = span_indices_batch[..., 0]
  en = span_indices_batch[..., 1]

  mesh = plsc.VectorSubcoreMesh(core_axis_name="c", subcore_axis_name="s")
  out_type = (
      jax.ShapeDtypeStruct((B, C), jnp.float32),
      jax.ShapeDtypeStruct((B, C), jnp.int32),
      jax.ShapeDtypeStruct((B, C), jnp.int32),
      jax.ShapeDtypeStruct((B, C), jnp.int32),
      jax.ShapeDtypeStruct((B, C), jnp.int32),
  )
  scratch = [
      pltpu.VMEM((C,), jnp.float32),
      pltpu.VMEM((C + 16,), jnp.float32),
      pltpu.VMEM((C,), jnp.int32),
      pltpu.VMEM((C,), jnp.int32),
      pltpu.VMEM((C,), jnp.int32),
      pltpu.VMEM((C,), jnp.int32),
      pltpu.VMEM((NSPAN,), jnp.int32),
      pltpu.VMEM((NSPAN,), jnp.int32),
      pltpu.VMEM((C,), jnp.int32),
      pltpu.VMEM((C,), jnp.int32),
      pltpu.VMEM((C,), jnp.int32),
      pltpu.VMEM((C,), jnp.float32),
      pltpu.VMEM((C,), jnp.int32),
  ] + [pltpu.VMEM((NBIN,), jnp.int32) for _ in range(2 * NQ)] + [
      pltpu.VMEM((512,), jnp.int32),
      pltpu.VMEM((512,), jnp.int32),
      pltpu.VMEM((512,), jnp.int32),
      pltpu.SemaphoreType.DMA,
  ]
  f = pl.kernel(_nms_body, out_type=out_type, mesh=mesh,
                scratch_types=scratch,
                compiler_params=pltpu.CompilerParams(
                    needs_layout_passes=False))
  ks, kp, ss, es, lb = f(sc, st, en)
  keep = kp.astype(bool)
  sp = jnp.stack([ss, es], axis=-1)
  return ks, keep, sp, lb


# speedup vs baseline: 482.8745x; 1.1034x over previous
import jax
import jax.numpy as jnp
from jax import lax
from jax.experimental import pallas as pl
from jax.experimental.pallas import tpu as pltpu
from jax.experimental.pallas import tpu_sc as plsc

B = 4
C = 8000
NV = C // 16
NBIN = 1024
NSPAN = 1000
THR = 0.5
KEY_BIAS = 0x7FFFFFFF


def _nms_body(sc_hbm, st_hbm, en_hbm,
              ks_hbm, kp_hbm, ss_hbm, es_hbm, lb_hbm, shx_hbm,
              sc_v, scs_v, key_a, idx_a, key_b, idx_b,
              st_in, en_in, ss_v, es_v, lb_v, ks_v, kp_v, stage,
              hist, pos, cov, prefE, pref2, nsm, sem):
  cax = lax.axis_index("c")
  sax = lax.axis_index("s")
  lane = lax.iota(jnp.int32, 16)
  active = sax < 4
  b = cax * 2 + lax.div(sax, 2)
  role = lax.rem(sax, 2)
  slotb = lax.div(sax, 2)

  @pl.when(active)
  def _():
    pltpu.sync_copy(sc_hbm.at[b], sc_v)
    pltpu.sync_copy(st_hbm.at[b], st_in)
    pltpu.sync_copy(en_hbm.at[b], en_in)

    take1 = role == 1

    def part_i(i, cur):
      s = sc_v[pl.ds(i * 16, 16)]
      m = jnp.logical_xor(s > THR, take1)
      k = KEY_BIAS - lax.bitcast_convert_type(s, jnp.int32)
      plsc.store_compressed(key_a.at[pl.ds(cur, 16)], k, mask=m)
      plsc.store_compressed(idx_a.at[pl.ds(cur, 16)], lane + i * 16, mask=m)
      return cur + plsc.all_reduce_population_count(m)[0]
    nloc = lax.fori_loop(0, NV, part_i, jnp.int32(0))
    n0 = jnp.where(role == 0, nloc, C - nloc)
    base2 = jnp.where(role == 0, 0, n0)
    nv = lax.div(nloc + 15, 16)
    nsm[0] = nloc

    for p, (sk, si, dk, di) in enumerate(
        [(key_a, idx_a, key_b, idx_b),
         (key_b, idx_b, key_a, idx_a),
         (key_a, idx_a, key_b, idx_b)]):
      shift = 10 * p

      @plsc.parallel_loop(0, NBIN, 16, unroll=4)
      def zero_h(j):
        hist[pl.ds(j, 16)] = jnp.zeros((16,), jnp.int32)

      def hist_i(i, _, sk=sk, shift=shift):
        k = sk[pl.ds(i * 16, 16)]
        m = (i * 16 + lane) < nloc
        d = lax.shift_right_logical(k, shift) & (NBIN - 1)
        cnt, last = plsc.scan_count(d, mask=m)
        plsc.addupdate_scatter(hist, [d], cnt, mask=last)
        return 0
      lax.fori_loop(0, nv, hist_i, 0)

      def scan_h(j, carry):
        h = hist[pl.ds(j * 16, 16)]
        c = plsc.cumsum(h)
        pos[pl.ds(j * 16, 16)] = c - h + carry
        return carry + jnp.max(c)
      lax.fori_loop(0, NBIN // 16, scan_h, jnp.int32(0))

      def perm_i(i, _, p=p, shift=shift, sk=sk, si=si, dk=dk, di=di):
        k = sk[pl.ds(i * 16, 16)]
        ii = si[pl.ds(i * 16, 16)]
        m = (i * 16 + lane) < nloc
        d = lax.shift_right_logical(k, shift) & (NBIN - 1)
        cnt, last = plsc.scan_count(d, mask=m)
        base = plsc.load_gather(pos, [d])
        tgt = base + cnt - 1
        if p == 2:
          tgt = tgt + base2
        else:
          plsc.store_scatter(dk, [tgt], k, mask=m)
        plsc.store_scatter(di, [tgt], ii, mask=m)
        plsc.addupdate_scatter(pos, [d], cnt, mask=last)
        return 0
      lax.fori_loop(0, nv, perm_i, 0)

    lo_v = lax.div(base2, 16)
    hi_v = lax.div(base2 + nloc + 15, 16)

    def post_i(v, _):
      i = v * 16
      ii = jnp.clip(idx_b[pl.ds(i, 16)], 0, C - 1)
      sidx = lax.shift_right_logical(ii, 3)
      ss_v[pl.ds(i, 16)] = plsc.load_gather(st_in, [sidx])
      es_v[pl.ds(i, 16)] = plsc.load_gather(en_in, [sidx])
      lb_v[pl.ds(i, 16)] = ii & 7
      scs_v[pl.ds(i, 16)] = plsc.load_gather(sc_v, [ii])
      return 0
    lax.fori_loop(lo_v, hi_v, post_i, 0)

    @pl.when(role == 0)
    def _():
      @plsc.parallel_loop(0, C, 16, unroll=4)
      def zero_out(i):
        ks_v[pl.ds(i, 16)] = jnp.zeros((16,), jnp.float32)
        kp_v[pl.ds(i, 16)] = jnp.zeros((16,), jnp.int32)

      @plsc.parallel_loop(0, 512, 16, unroll=4)
      def zero_cp(j):
        cov[pl.ds(j, 16)] = jnp.zeros((16,), jnp.int32)
        prefE[pl.ds(j, 16)] = jnp.zeros((16,), jnp.int32)
        pref2[pl.ds(j, 16)] = jnp.zeros((16,), jnp.int32)

    @pl.when(role == 1)
    def _():
      pltpu.sync_copy(ss_v, shx_hbm.at[b * 3])
      pltpu.sync_copy(es_v, shx_hbm.at[b * 3 + 1])
      pltpu.sync_copy(lb_v, shx_hbm.at[b * 3 + 2])

  plsc.subcore_barrier()

  @pl.when(jnp.logical_and(active, role == 0))
  def _():
    count = nsm[0]

    for a, dstv in enumerate([ss_v, es_v, lb_v]):
      pltpu.sync_copy(shx_hbm.at[b * 3 + a], stage)

      def mrg(v, _, dstv=dstv):
        i = v * 16
        m = (i + lane) >= count
        dstv[pl.ds(i, 16)] = jnp.where(m, stage[pl.ds(i, 16)],
                                       dstv[pl.ds(i, 16)])
        return 0
      lax.fori_loop(lax.div(count, 16), NV, mrg, 0)

    ngroups = lax.div(count + 15, 16)

    def g_body(g, _):
      off = g * 16
      valid = (off + lane) < count
      st = ss_v[pl.ds(off, 16)]
      en = es_v[pl.ds(off, 16)]

      def i_cond(ic):
        return ic[1]

      def i_body(ic):
        prev_k, _ = ic
        pfs = plsc.load_gather(pref2, [st])
        pfe = plsc.load_gather(prefE, [en])
        conf = (pfe - pfs) > 0
        cand = jnp.logical_and(
            jnp.logical_and(valid, jnp.logical_not(conf)), lane > prev_k)
        kidx = plsc.all_reduce_ffs(cand)[0]
        has = jnp.logical_and(kidx >= 0, kidx < 16)

        @pl.when(has)
        def _():
          onehot = lane == kidx
          s_k = jnp.max(jnp.where(onehot, st, jnp.int32(-1)))
          e_k = jnp.max(jnp.where(onehot, en, jnp.int32(-1)))

          def upd(j, carry):
            gpos = lane + j * 16
            cvj = cov[pl.ds(j * 16, 16)]
            m = jnp.logical_and(gpos >= s_k, gpos <= e_k)
            cvj = jnp.where(m, jnp.int32(1), cvj)
            cov[pl.ds(j * 16, 16)] = cvj
            cs = plsc.cumsum(cvj)
            prefE[pl.ds(j * 16, 16)] = cs + carry
            pref2[pl.ds(j * 16, 16)] = cs - cvj + carry
            return carry + jnp.max(cs)
          lax.fori_loop(0, 32, upd, jnp.int32(0))

          kv = kp_v[pl.ds(off, 16)] | jnp.where(onehot, jnp.int32(1),
                                                jnp.int32(0))
          kp_v[pl.ds(off, 16)] = kv
          scv = scs_v[pl.ds(off, 16)]
          ks_v[pl.ds(off, 16)] = scv * kv.astype(jnp.float32)

        prev2 = jnp.where(has, kidx, prev_k)
        return (prev2, has)

      lax.while_loop(i_cond, i_body, (jnp.int32(-1), True))
      return 0

    lax.fori_loop(0, ngroups, g_body, 0)

    d1 = pltpu.async_copy(ks_v, ks_hbm.at[b], sem)
    d2 = pltpu.async_copy(kp_v, kp_hbm.at[b], sem)
    d3 = pltpu.async_copy(ss_v, ss_hbm.at[b], sem)
    d4 = pltpu.async_copy(es_v, es_hbm.at[b], sem)
    d5 = pltpu.async_copy(lb_v, lb_hbm.at[b], sem)
    d1.wait()
    d2.wait()
    d3.wait()
    d4.wait()
    d5.wait()


def kernel(probs_batch, span_indices_batch):
  sc = probs_batch.reshape(B, C)
  st = span_indices_batch[..., 0]
  en = span_indices_batch[..., 1]

  mesh = plsc.VectorSubcoreMesh(core_axis_name="c", subcore_axis_name="s")
  out_type = (
      jax.ShapeDtypeStruct((B, C), jnp.float32),
      jax.ShapeDtypeStruct((B, C), jnp.int32),
      jax.ShapeDtypeStruct((B, C), jnp.int32),
      jax.ShapeDtypeStruct((B, C), jnp.int32),
      jax.ShapeDtypeStruct((B, C), jnp.int32),
      jax.ShapeDtypeStruct((B * 3, C), jnp.int32),
  )
  scratch = [
      pltpu.VMEM((C,), jnp.float32),
      pltpu.VMEM((C,), jnp.float32),
      pltpu.VMEM((C + 16,), jnp.int32),
      pltpu.VMEM((C + 16,), jnp.int32),
      pltpu.VMEM((C,), jnp.int32),
      pltpu.VMEM((C,), jnp.int32),
      pltpu.VMEM((NSPAN,), jnp.int32),
      pltpu.VMEM((NSPAN,), jnp.int32),
      pltpu.VMEM((C,), jnp.int32),
      pltpu.VMEM((C,), jnp.int32),
      pltpu.VMEM((C,), jnp.int32),
      pltpu.VMEM((C,), jnp.float32),
      pltpu.VMEM((C,), jnp.int32),
      pltpu.VMEM((C,), jnp.int32),
      pltpu.VMEM((NBIN,), jnp.int32),
      pltpu.VMEM((NBIN,), jnp.int32),
      pltpu.VMEM((512,), jnp.int32),
      pltpu.VMEM((512,), jnp.int32),
      pltpu.VMEM((512,), jnp.int32),
      pltpu.SMEM((8,), jnp.int32),
      pltpu.SemaphoreType.DMA,
  ]
  f = pl.kernel(_nms_body, out_type=out_type, mesh=mesh,
                scratch_types=scratch,
                compiler_params=pltpu.CompilerParams(
                    needs_layout_passes=False))
  ks, kp, ss, es, lb, _shx = f(sc, st, en)
  keep = kp.astype(bool)
  sp = jnp.stack([ss, es], axis=-1)
  return ks, keep, sp, lb
